# baseline (speedup 1.0000x reference)
import functools

import jax
import jax.numpy as jnp
from jax import lax
from jax.experimental import pallas as pl
from jax.experimental.pallas import tpu as pltpu
from jax.experimental.pallas import tpu_sc as plsc

NC = 2
NS = 16
NW = NC * NS
C = 128


@functools.partial(jax.jit, static_argnums=(2, 3))
def _sc_gather(idx_flat, table, n_per_w, total_rows):
    dim = table.shape[1]
    mesh = plsc.VectorSubcoreMesh(
        core_axis_name="c", subcore_axis_name="s",
        num_cores=NC, num_subcores=NS,
    )

    @functools.partial(
        pl.kernel,
        out_type=jax.ShapeDtypeStruct((total_rows, dim), jnp.float32),
        mesh=mesh,
        scratch_types=[
            pltpu.VMEM((n_per_w * C,), jnp.int32),
            pltpu.VMEM((C, dim), jnp.float32),
            pltpu.SemaphoreType.DMA,
        ],
        compiler_params=pltpu.CompilerParams(use_tc_tiling_on_sc=False),
    )
    def k(idx_hbm, table_hbm, out_hbm, idx_v, rows_v, sem):
        wid = lax.axis_index("s") * NC + lax.axis_index("c")
        base = wid * n_per_w * C
        pltpu.sync_copy(idx_hbm.at[pl.ds(base, n_per_w * C)], idx_v)

        @pl.loop(0, n_per_w)
        def _(j):
            pltpu.async_copy(
                table_hbm.at[idx_v.at[pl.ds(j * C, C)]], rows_v, sem
            ).wait()
            pltpu.sync_copy(rows_v, out_hbm.at[pl.ds(base + j * C, C)])

    return k(idx_flat, table)


def kernel(input_indices, weight):
    b, s = input_indices.shape
    total = b * s
    idx_flat = input_indices.reshape(total).astype(jnp.int32)
    n_per_w = (total // C) // NW
    out = _sc_gather(idx_flat, weight, n_per_w, total)
    return out.reshape(b, s, weight.shape[1])


# speedup vs baseline: 1.0446x; 1.0446x over previous
import functools

import jax
import jax.numpy as jnp
from jax import lax
from jax.experimental import pallas as pl
from jax.experimental.pallas import tpu as pltpu
from jax.experimental.pallas import tpu_sc as plsc

NC = 2
NS = 16
NW = NC * NS
C = 128


@functools.partial(jax.jit, static_argnums=(2, 3))
def _sc_gather(idx_flat, table, n_per_w, total_rows):
    dim = table.shape[1]
    mesh = plsc.VectorSubcoreMesh(
        core_axis_name="c", subcore_axis_name="s",
        num_cores=NC, num_subcores=NS,
    )

    K = 25
    n_groups = n_per_w // K

    @functools.partial(
        pl.kernel,
        out_type=jax.ShapeDtypeStruct((total_rows, dim), jnp.float32),
        mesh=mesh,
        scratch_types=[
            pltpu.VMEM((n_per_w * C,), jnp.int32),
            pltpu.VMEM((K * C, dim), jnp.float32),
            pltpu.SemaphoreType.DMA,
        ],
        compiler_params=pltpu.CompilerParams(use_tc_tiling_on_sc=False),
    )
    def k(idx_hbm, table_hbm, out_hbm, idx_v, rows_v, sem):
        wid = lax.axis_index("s") * NC + lax.axis_index("c")
        base = wid * n_per_w * C
        pltpu.sync_copy(idx_hbm.at[pl.ds(base, n_per_w * C)], idx_v)

        @pl.loop(0, n_groups)
        def _(g):
            @pl.loop(0, K)
            def _(j):
                pltpu.async_copy(
                    table_hbm.at[idx_v.at[pl.ds((g * K + j) * C, C)]],
                    rows_v.at[pl.ds(j * C, C)],
                    sem,
                )

            pltpu.make_async_copy(
                table_hbm.at[pl.ds(0, K * C)], rows_v, sem
            ).wait()
            pltpu.sync_copy(
                rows_v, out_hbm.at[pl.ds(base + g * K * C, K * C)]
            )

    return k(idx_flat, table)


def kernel(input_indices, weight):
    b, s = input_indices.shape
    total = b * s
    idx_flat = input_indices.reshape(total).astype(jnp.int32)
    n_per_w = (total // C) // NW
    out = _sc_gather(idx_flat, weight, n_per_w, total)
    return out.reshape(b, s, weight.shape[1])
